# baseline (speedup 1.0000x reference)
import jax
import jax.numpy as jnp
from jax import lax
from jax.experimental import pallas as pl
from jax.experimental.pallas import tpu as pltpu
from jax.experimental.pallas import tpu_sc as plsc

N_PTS = 262144
R = 64
D = 32
NW = 32
PER_W = N_PTS // NW
C = 128
N_CHUNKS = PER_W // C
NHALF = N_CHUNKS // 2
G = C // 16
ROWS = 8 * C
GB = 1024
NGATHER = ROWS // GB
OUT_ROWS = C * D // 128
BLK = 4 * C

_C_OFF = (0, 1, 65, 66, 4225, 4226, 4290, 4291)


def _body(blk_hbm, emb_hbm, out_hbm,
          in0, in1, cidx0, cidx1, emb0, emb1, out0, out1,
          sem0, sem1, semo0, semo1):
    wid = lax.axis_index("s") * 2 + lax.axis_index("c")
    vs = 2.0 / R
    inv_vs = R / 2.0

    def load_and_issue(c, in_v, cidx_v, emb_v, sem):
        k = wid * N_CHUNKS + c
        pltpu.sync_copy(blk_hbm.at[pl.ds(k * BLK, BLK)], in_v)

        def idx_body(g, _):
            vox = in_v[pl.ds(g * 16, 16)].astype(jnp.int32)
            vi = lax.shift_right_logical(vox, 12)
            vj = lax.bitwise_and(lax.shift_right_logical(vox, 6), 63)
            vk = lax.bitwise_and(vox, 63)
            cb = vi * 4225 + vj * 65 + vk
            for j in range(8):
                cidx_v[pl.ds(j * C + g * 16, 16)] = cb + _C_OFF[j]
            return _

        lax.fori_loop(0, G, idx_body, None)
        for b in range(NGATHER):
            pltpu.async_copy(
                emb_hbm.at[cidx_v.at[pl.ds(b * GB, GB)]],
                emb_v.at[pl.ds(b * GB, GB)], sem)

    def drain_gathers(emb_v, sem):
        pltpu.make_async_copy(emb_hbm.at[pl.ds(0, ROWS)], emb_v, sem).wait()

    def combine_and_store(c, in_v, emb_v, out_v, semo, first):
        base = wid * PER_W + c * C

        @pl.when(jnp.logical_not(first))
        def _drain_prev_store():
            pltpu.make_async_copy(
                out_v, out_hbm.at[pl.ds(0, C * D)], semo).wait()

        def acc_body(g, _):
            vox = in_v[pl.ds(g * 16, 16)].astype(jnp.int32)
            vi = lax.shift_right_logical(vox, 12)
            vj = lax.bitwise_and(lax.shift_right_logical(vox, 6), 63)
            vk = lax.bitwise_and(vox, 63)
            px = in_v[pl.ds(C + g * 16, 16)]
            py = in_v[pl.ds(2 * C + g * 16, 16)]
            pz = in_v[pl.ds(3 * C + g * 16, 16)]
            cx = (vi.astype(jnp.float32) + 0.5) * vs - 1.0
            cy = (vj.astype(jnp.float32) + 0.5) * vs - 1.0
            cz = (vk.astype(jnp.float32) + 0.5) * vs - 1.0
            x = (px - cx) * inv_vs + 0.5
            y = (py - cy) * inv_vs + 0.5
            z = (pz - cz) * inv_vs + 0.5
            x0, y0, z0 = 1.0 - x, 1.0 - y, 1.0 - z
            wxy = (x0 * y0, x0 * y, x * y0, x * y)
            w = [wxy[j >> 1] * (z if (j & 1) else z0) for j in range(8)]
            for t in range(16):
                n = g * 16 + t
                acc0 = jnp.zeros((16,), jnp.float32)
                acc1 = jnp.zeros((16,), jnp.float32)
                for j in range(8):
                    row = j * C + n
                    wj = w[j][t]
                    acc0 = acc0 + emb_v[row, pl.ds(0, 16)] * wj
                    acc1 = acc1 + emb_v[row, pl.ds(16, 16)] * wj
                out_v[pl.ds(n * D, 16)] = acc0
                out_v[pl.ds(n * D + 16, 16)] = acc1
            return _

        lax.fori_loop(0, G, acc_body, None)
        pltpu.async_copy(
            out_v, out_hbm.at[pl.ds(base * D, C * D)], semo)

    load_and_issue(0, in0, cidx0, emb0, sem0)

    def pipe_body(i, _):
        c0 = 2 * i
        load_and_issue(c0 + 1, in1, cidx1, emb1, sem1)
        drain_gathers(emb0, sem0)
        combine_and_store(c0, in0, emb0, out0, semo0, i == 0)

        @pl.when(i < NHALF - 1)
        def _stage_next():
            load_and_issue(c0 + 2, in0, cidx0, emb0, sem0)

        drain_gathers(emb1, sem1)
        combine_and_store(c0 + 1, in1, emb1, out1, semo1, i == 0)
        return _

    lax.fori_loop(0, NHALF, pipe_body, None)

    pltpu.make_async_copy(out0, out_hbm.at[pl.ds(0, C * D)], semo0).wait()
    pltpu.make_async_copy(out1, out_hbm.at[pl.ds(0, C * D)], semo1).wait()


@jax.jit
def _run(pts, p2v_idx, voxel_embeddings):
    mesh = plsc.VectorSubcoreMesh(core_axis_name="c", subcore_axis_name="s")
    f = pl.kernel(
        _body,
        out_type=jax.ShapeDtypeStruct((N_PTS * D,), jnp.float32),
        mesh=mesh,
        compiler_params=pltpu.CompilerParams(use_tc_tiling_on_sc=False),
        scratch_types=[
            pltpu.VMEM((BLK,), jnp.float32),
            pltpu.VMEM((BLK,), jnp.float32),
            pltpu.VMEM((ROWS,), jnp.int32),
            pltpu.VMEM((ROWS,), jnp.int32),
            pltpu.VMEM((ROWS, D), jnp.float32),
            pltpu.VMEM((ROWS, D), jnp.float32),
            pltpu.VMEM((C * D,), jnp.float32),
            pltpu.VMEM((C * D,), jnp.float32),
            pltpu.SemaphoreType.DMA,
            pltpu.SemaphoreType.DMA,
            pltpu.SemaphoreType.DMA,
            pltpu.SemaphoreType.DMA,
        ],
    )
    quad = jnp.stack([p2v_idx.astype(jnp.float32),
                      pts[:, 0], pts[:, 1], pts[:, 2]], axis=0)
    blk = quad.reshape(4, N_PTS // C, C).transpose(1, 0, 2).reshape(-1)
    out = f(blk, voxel_embeddings)
    return out.reshape(N_PTS, D)


def kernel(pts, p2v_idx, center_points, corner_points, center2corner,
           voxel_embeddings, interp_offset, voxel_size):
    return _run(pts, p2v_idx, voxel_embeddings)


# speedup vs baseline: 1.0013x; 1.0013x over previous
import jax
import jax.numpy as jnp
from jax import lax
from jax.experimental import pallas as pl
from jax.experimental.pallas import tpu as pltpu
from jax.experimental.pallas import tpu_sc as plsc

N_PTS = 262144
R = 64
D = 32
NW = 32
PER_W = N_PTS // NW
C = 128
N_CHUNKS = PER_W // C
NHALF = N_CHUNKS // 2
G = C // 16
ROWS = 8 * C
GB = 128
NGATHER = ROWS // GB
OUT_ROWS = C * D // 128
BLK = 4 * C

_C_OFF = (0, 1, 65, 66, 4225, 4226, 4290, 4291)


def _body(blk_hbm, emb_hbm, out_hbm,
          in0, in1, cidx0, cidx1, emb0, emb1, out0, out1,
          sem0, sem1, semo0, semo1):
    wid = lax.axis_index("s") * 2 + lax.axis_index("c")
    vs = 2.0 / R
    inv_vs = R / 2.0

    def load_and_issue(c, in_v, cidx_v, emb_v, sem):
        k = wid * N_CHUNKS + c
        pltpu.sync_copy(blk_hbm.at[pl.ds(k * BLK, BLK)], in_v)

        def idx_body(g, _):
            vox = in_v[pl.ds(g * 16, 16)].astype(jnp.int32)
            vi = lax.shift_right_logical(vox, 12)
            vj = lax.bitwise_and(lax.shift_right_logical(vox, 6), 63)
            vk = lax.bitwise_and(vox, 63)
            cb = vi * 4225 + vj * 65 + vk
            for j in range(8):
                cidx_v[pl.ds(j * C + g * 16, 16)] = cb + _C_OFF[j]
            return _

        lax.fori_loop(0, G, idx_body, None)
        for b in range(NGATHER):
            pltpu.async_copy(
                emb_hbm.at[cidx_v.at[pl.ds(b * GB, GB)]],
                emb_v.at[pl.ds(b * GB, GB)], sem)

    def drain_gathers(emb_v, sem):
        pltpu.make_async_copy(emb_hbm.at[pl.ds(0, ROWS)], emb_v, sem).wait()

    def combine_and_store(c, in_v, emb_v, out_v, semo, first):
        base = wid * PER_W + c * C

        @pl.when(jnp.logical_not(first))
        def _drain_prev_store():
            pltpu.make_async_copy(
                out_v, out_hbm.at[pl.ds(0, C * D)], semo).wait()

        def acc_body(g, _):
            vox = in_v[pl.ds(g * 16, 16)].astype(jnp.int32)
            vi = lax.shift_right_logical(vox, 12)
            vj = lax.bitwise_and(lax.shift_right_logical(vox, 6), 63)
            vk = lax.bitwise_and(vox, 63)
            px = in_v[pl.ds(C + g * 16, 16)]
            py = in_v[pl.ds(2 * C + g * 16, 16)]
            pz = in_v[pl.ds(3 * C + g * 16, 16)]
            cx = (vi.astype(jnp.float32) + 0.5) * vs - 1.0
            cy = (vj.astype(jnp.float32) + 0.5) * vs - 1.0
            cz = (vk.astype(jnp.float32) + 0.5) * vs - 1.0
            x = (px - cx) * inv_vs + 0.5
            y = (py - cy) * inv_vs + 0.5
            z = (pz - cz) * inv_vs + 0.5
            x0, y0, z0 = 1.0 - x, 1.0 - y, 1.0 - z
            wxy = (x0 * y0, x0 * y, x * y0, x * y)
            w = [wxy[j >> 1] * (z if (j & 1) else z0) for j in range(8)]
            for t in range(16):
                n = g * 16 + t
                acc0 = jnp.zeros((16,), jnp.float32)
                acc1 = jnp.zeros((16,), jnp.float32)
                for j in range(8):
                    row = j * C + n
                    wj = w[j][t]
                    acc0 = acc0 + emb_v[row, pl.ds(0, 16)] * wj
                    acc1 = acc1 + emb_v[row, pl.ds(16, 16)] * wj
                out_v[pl.ds(n * D, 16)] = acc0
                out_v[pl.ds(n * D + 16, 16)] = acc1
            return _

        lax.fori_loop(0, G, acc_body, None)
        pltpu.async_copy(
            out_v, out_hbm.at[pl.ds(base * D, C * D)], semo)

    load_and_issue(0, in0, cidx0, emb0, sem0)

    def pipe_body(i, _):
        c0 = 2 * i
        load_and_issue(c0 + 1, in1, cidx1, emb1, sem1)
        drain_gathers(emb0, sem0)
        combine_and_store(c0, in0, emb0, out0, semo0, i == 0)

        @pl.when(i < NHALF - 1)
        def _stage_next():
            load_and_issue(c0 + 2, in0, cidx0, emb0, sem0)

        drain_gathers(emb1, sem1)
        combine_and_store(c0 + 1, in1, emb1, out1, semo1, i == 0)
        return _

    lax.fori_loop(0, NHALF, pipe_body, None)

    pltpu.make_async_copy(out0, out_hbm.at[pl.ds(0, C * D)], semo0).wait()
    pltpu.make_async_copy(out1, out_hbm.at[pl.ds(0, C * D)], semo1).wait()


@jax.jit
def _run(pts, p2v_idx, voxel_embeddings):
    mesh = plsc.VectorSubcoreMesh(core_axis_name="c", subcore_axis_name="s")
    f = pl.kernel(
        _body,
        out_type=jax.ShapeDtypeStruct((N_PTS * D,), jnp.float32),
        mesh=mesh,
        compiler_params=pltpu.CompilerParams(use_tc_tiling_on_sc=False),
        scratch_types=[
            pltpu.VMEM((BLK,), jnp.float32),
            pltpu.VMEM((BLK,), jnp.float32),
            pltpu.VMEM((ROWS,), jnp.int32),
            pltpu.VMEM((ROWS,), jnp.int32),
            pltpu.VMEM((ROWS, D), jnp.float32),
            pltpu.VMEM((ROWS, D), jnp.float32),
            pltpu.VMEM((C * D,), jnp.float32),
            pltpu.VMEM((C * D,), jnp.float32),
            pltpu.SemaphoreType.DMA,
            pltpu.SemaphoreType.DMA,
            pltpu.SemaphoreType.DMA,
            pltpu.SemaphoreType.DMA,
        ],
    )
    quad = jnp.stack([p2v_idx.astype(jnp.float32),
                      pts[:, 0], pts[:, 1], pts[:, 2]], axis=0)
    blk = quad.reshape(4, N_PTS // C, C).transpose(1, 0, 2).reshape(-1)
    out = f(blk, voxel_embeddings)
    return out.reshape(N_PTS, D)


def kernel(pts, p2v_idx, center_points, corner_points, center2corner,
           voxel_embeddings, interp_offset, voxel_size):
    return _run(pts, p2v_idx, voxel_embeddings)
